# baseline (speedup 1.0000x reference)
import functools

import jax
import jax.numpy as jnp
from jax import lax
from jax.experimental import pallas as pl
from jax.experimental.pallas import tpu as pltpu
from jax.experimental.pallas import tpu_sc as plsc

_CPARAMS = pltpu.CompilerParams(needs_layout_passes=False)

_PCTS = (0.25, 0.5, 0.75)
_L = 16
_NC = 2
_NS = 16
_NW = _NC * _NS
_WIN = 12


def _f32v(v):
    return jnp.full((_L,), v, jnp.float32)


def _i32v(v):
    return jnp.full((_L,), v, jnp.int32)


@functools.lru_cache(maxsize=None)
def _build_fast(Bn):
    RW = Bn // _NW
    G = RW // _L
    mesh = plsc.VectorSubcoreMesh(core_axis_name="c", subcore_axis_name="s")

    @functools.partial(
        pl.kernel,
        mesh=mesh,
        out_type=jax.ShapeDtypeStruct((_NW * 2 * _L,), jnp.float32),
        scratch_types=[
            pltpu.VMEM((RW * _WIN,), jnp.float32),
            pltpu.VMEM((RW * _WIN,), jnp.float32),
            pltpu.VMEM((_WIN * _L,), jnp.float32),
            pltpu.VMEM((_WIN * _L,), jnp.float32),
            pltpu.VMEM((2 * _L,), jnp.float32),
        ],
        compiler_params=_CPARAMS,
    )
    def fast(we_hbm, wt_hbm, out_acc, tl_t, tl_e, cf_t, cf_e, v01):
        wid = lax.axis_index("s") * _NC + lax.axis_index("c")
        base = wid * RW * _WIN
        pltpu.sync_copy(wt_hbm.at[pl.ds(base, RW * _WIN)], tl_t)
        pltpu.sync_copy(we_hbm.at[pl.ds(base, RW * _WIN)], tl_e)
        lane = lax.iota(jnp.int32, _L)

        def group(g, carry):
            acc, unres = carry
            rowbase = (g * _L + lane) * _WIN
            run_t = jnp.zeros((_L,), jnp.float32)
            run_e = jnp.zeros((_L,), jnp.float32)
            cnt_t = [jnp.zeros((_L,), jnp.int32) for _ in _PCTS]
            cnt_e = [jnp.zeros((_L,), jnp.int32) for _ in _PCTS]
            for j in range(_WIN):
                cols = rowbase + _i32v(j)
                x_t = plsc.load_gather(tl_t, [cols])
                x_e = plsc.load_gather(tl_e, [cols])
                run_t = run_t + x_t
                run_e = run_e + x_e
                cf_t[pl.ds(j * _L, _L)] = run_t
                cf_e[pl.ds(j * _L, _L)] = run_e
                for k, p in enumerate(_PCTS):
                    cnt_t[k] = cnt_t[k] + (run_t < _f32v(p)).astype(jnp.int32)
                    cnt_e[k] = cnt_e[k] + (run_e < _f32v(p)).astype(jnp.int32)

            def interp(tile, cdft, cnt, p):
                idx = jnp.minimum(cnt, _i32v(_WIN - 1))
                yb = plsc.load_gather(cdft, [idx * _L + lane])
                aa = plsc.load_gather(tile, [rowbase + idx])
                return idx.astype(jnp.float32) + _f32v(1.0) + (_f32v(p) - yb) / aa

            for k, p in enumerate(_PCTS):
                q_t = interp(tl_t, cf_t, cnt_t[k], p)
                q_e = interp(tl_e, cf_e, cnt_e[k], p)
                acc = acc + jnp.abs(q_t - q_e)
            u = (cnt_t[2] >= _i32v(_WIN)) | (cnt_e[2] >= _i32v(_WIN))
            unres = unres + jnp.where(u, _f32v(1.0), _f32v(0.0))
            return acc, unres

        acc, unres = lax.fori_loop(
            0, G, group,
            (jnp.zeros((_L,), jnp.float32), jnp.zeros((_L,), jnp.float32)))
        v01[pl.ds(0, _L)] = acc
        v01[pl.ds(_L, _L)] = unres
        pltpu.sync_copy(v01, out_acc.at[pl.ds(wid * 2 * _L, 2 * _L)])

    return fast


@functools.lru_cache(maxsize=None)
def _build_full(Bn, Nn):
    RW = Bn // _NW
    G = RW // _L
    mesh = plsc.VectorSubcoreMesh(core_axis_name="c", subcore_axis_name="s")

    @functools.partial(
        pl.kernel,
        mesh=mesh,
        out_type=jax.ShapeDtypeStruct((_NW * _L,), jnp.float32),
        scratch_types=[
            pltpu.VMEM((_L * Nn,), jnp.float32),
            pltpu.VMEM((_L * Nn,), jnp.float32),
            pltpu.VMEM((Nn * _L,), jnp.float32),
            pltpu.VMEM((Nn * _L,), jnp.float32),
            pltpu.VMEM((_L,), jnp.float32),
        ],
        compiler_params=_CPARAMS,
    )
    def full(pe_hbm, pt_hbm, out_acc, tl_t, tl_e, cf_t, cf_e, v0):
        wid = lax.axis_index("s") * _NC + lax.axis_index("c")
        base = wid * RW * Nn
        lane = lax.iota(jnp.int32, _L)

        def scan_tile(tile, cdft):
            rowbase = lane * Nn

            def col(j, carry):
                run, c0, c1, c2 = carry
                x = plsc.load_gather(tile, [rowbase + j])
                run = run + x
                plsc.store_scatter(cdft, [j * _L + lane], run)
                c0 = c0 + jnp.where(run < _f32v(0.25), _i32v(1), _i32v(0))
                c1 = c1 + jnp.where(run < _f32v(0.5), _i32v(1), _i32v(0))
                c2 = c2 + jnp.where(run < _f32v(0.75), _i32v(1), _i32v(0))
                return run, c0, c1, c2

            z = jnp.zeros((_L,), jnp.int32)
            _, c0, c1, c2 = lax.fori_loop(
                0, Nn, col, (jnp.zeros((_L,), jnp.float32), z, z, z))
            qs = []
            for p, cnt in zip(_PCTS, (c0, c1, c2)):
                idx = jnp.where(cnt >= _i32v(Nn), _i32v(0), cnt)
                yb = plsc.load_gather(cdft, [idx * _L + lane])
                aa = plsc.load_gather(tile, [rowbase + idx])
                qs.append(idx.astype(jnp.float32) + _f32v(1.0) + (_f32v(p) - yb) / aa)
            return qs

        def group(g, acc):
            off = base + g * _L * Nn
            pltpu.sync_copy(pt_hbm.at[pl.ds(off, _L * Nn)], tl_t)
            pltpu.sync_copy(pe_hbm.at[pl.ds(off, _L * Nn)], tl_e)
            q_t = scan_tile(tl_t, cf_t)
            q_e = scan_tile(tl_e, cf_e)
            for k in range(len(_PCTS)):
                acc = acc + jnp.abs(q_t[k] - q_e[k])
            return acc

        acc = lax.fori_loop(0, G, group, jnp.zeros((_L,), jnp.float32))
        v0[...] = acc
        pltpu.sync_copy(v0, out_acc.at[pl.ds(wid * _L, _L)])

    return full


def kernel(p_estimate, p_target):
    Bn, Nn = p_estimate.shape
    we = lax.slice(p_estimate, (0, 0), (Bn, _WIN)).reshape(-1)
    wt = lax.slice(p_target, (0, 0), (Bn, _WIN)).reshape(-1)
    accu = _build_fast(Bn)(we, wt).reshape(_NW, 2, _L)
    loss_fast = jnp.sum(accu[:, 0]) / jnp.float32(Bn * len(_PCTS))
    unres = accu[:, 1]

    def _full_path(ops):
        pe, pt = ops
        facc = _build_full(Bn, Nn)(pe.reshape(-1), pt.reshape(-1))
        return jnp.sum(facc) / jnp.float32(Bn * len(_PCTS))

    def _fast_path(ops):
        return loss_fast

    return lax.cond(jnp.sum(unres) > 0.0, _full_path, _fast_path,
                    (p_estimate, p_target))


# speedup vs baseline: 1.0335x; 1.0335x over previous
import functools

import jax
import jax.numpy as jnp
from jax import lax
from jax.experimental import pallas as pl
from jax.experimental.pallas import tpu as pltpu
from jax.experimental.pallas import tpu_sc as plsc

_CPARAMS = pltpu.CompilerParams(needs_layout_passes=False)

_PCTS = (0.25, 0.5, 0.75)
_L = 16
_NC = 2
_NS = 16
_NW = _NC * _NS
_WIN = 16


def _f32v(v):
    return jnp.full((_L,), v, jnp.float32)


def _i32v(v):
    return jnp.full((_L,), v, jnp.int32)


@functools.lru_cache(maxsize=None)
def _build_fast(Bn):
    RW = Bn // _NW
    G = RW // _L
    mesh = plsc.VectorSubcoreMesh(core_axis_name="c", subcore_axis_name="s")

    @functools.partial(
        pl.kernel,
        mesh=mesh,
        out_type=jax.ShapeDtypeStruct((_NW * 2 * _L,), jnp.float32),
        scratch_types=[
            pltpu.VMEM((RW * _WIN,), jnp.float32),
            pltpu.VMEM((RW * _WIN,), jnp.float32),
            pltpu.VMEM((_WIN * _L,), jnp.float32),
            pltpu.VMEM((_WIN * _L,), jnp.float32),
            pltpu.VMEM((2 * _L,), jnp.float32),
        ],
        compiler_params=_CPARAMS,
    )
    def fast(we_hbm, wt_hbm, out_acc, tl_t, tl_e, cf_t, cf_e, v01):
        wid = lax.axis_index("s") * _NC + lax.axis_index("c")
        base = wid * RW * _WIN
        pltpu.sync_copy(wt_hbm.at[pl.ds(base, RW * _WIN)], tl_t)
        pltpu.sync_copy(we_hbm.at[pl.ds(base, RW * _WIN)], tl_e)
        lane = lax.iota(jnp.int32, _L)

        def group(g, carry):
            acc, unres = carry
            rowbase = (g * _L + lane) * _WIN
            run_t = jnp.zeros((_L,), jnp.float32)
            run_e = jnp.zeros((_L,), jnp.float32)
            cnt_t = [jnp.zeros((_L,), jnp.int32) for _ in _PCTS]
            cnt_e = [jnp.zeros((_L,), jnp.int32) for _ in _PCTS]
            for j in range(_WIN):
                cols = rowbase + _i32v(j)
                x_t = plsc.load_gather(tl_t, [cols])
                x_e = plsc.load_gather(tl_e, [cols])
                run_t = run_t + x_t
                run_e = run_e + x_e
                cf_t[pl.ds(j * _L, _L)] = run_t
                cf_e[pl.ds(j * _L, _L)] = run_e
                for k, p in enumerate(_PCTS):
                    cnt_t[k] = cnt_t[k] + (run_t < _f32v(p)).astype(jnp.int32)
                    cnt_e[k] = cnt_e[k] + (run_e < _f32v(p)).astype(jnp.int32)

            def interp(tile, cdft, cnt, p):
                idx = jnp.minimum(cnt, _i32v(_WIN - 1))
                yb = plsc.load_gather(cdft, [idx * _L + lane])
                aa = plsc.load_gather(tile, [rowbase + idx])
                return idx.astype(jnp.float32) + _f32v(1.0) + (_f32v(p) - yb) / aa

            for k, p in enumerate(_PCTS):
                q_t = interp(tl_t, cf_t, cnt_t[k], p)
                q_e = interp(tl_e, cf_e, cnt_e[k], p)
                acc = acc + jnp.abs(q_t - q_e)
            u = (cnt_t[2] >= _i32v(_WIN)) | (cnt_e[2] >= _i32v(_WIN))
            unres = unres + jnp.where(u, _f32v(1.0), _f32v(0.0))
            return acc, unres

        acc, unres = lax.fori_loop(
            0, G, group,
            (jnp.zeros((_L,), jnp.float32), jnp.zeros((_L,), jnp.float32)))
        v01[pl.ds(0, _L)] = acc
        v01[pl.ds(_L, _L)] = unres
        pltpu.sync_copy(v01, out_acc.at[pl.ds(wid * 2 * _L, 2 * _L)])

    return fast


@functools.lru_cache(maxsize=None)
def _build_full(Bn, Nn):
    RW = Bn // _NW
    G = RW // _L
    mesh = plsc.VectorSubcoreMesh(core_axis_name="c", subcore_axis_name="s")

    @functools.partial(
        pl.kernel,
        mesh=mesh,
        out_type=jax.ShapeDtypeStruct((_NW * _L,), jnp.float32),
        scratch_types=[
            pltpu.VMEM((_L * Nn,), jnp.float32),
            pltpu.VMEM((_L * Nn,), jnp.float32),
            pltpu.VMEM((Nn * _L,), jnp.float32),
            pltpu.VMEM((Nn * _L,), jnp.float32),
            pltpu.VMEM((_L,), jnp.float32),
        ],
        compiler_params=_CPARAMS,
    )
    def full(pe_hbm, pt_hbm, out_acc, tl_t, tl_e, cf_t, cf_e, v0):
        wid = lax.axis_index("s") * _NC + lax.axis_index("c")
        base = wid * RW * Nn
        lane = lax.iota(jnp.int32, _L)

        def scan_tile(tile, cdft):
            rowbase = lane * Nn

            def col(j, carry):
                run, c0, c1, c2 = carry
                x = plsc.load_gather(tile, [rowbase + j])
                run = run + x
                plsc.store_scatter(cdft, [j * _L + lane], run)
                c0 = c0 + jnp.where(run < _f32v(0.25), _i32v(1), _i32v(0))
                c1 = c1 + jnp.where(run < _f32v(0.5), _i32v(1), _i32v(0))
                c2 = c2 + jnp.where(run < _f32v(0.75), _i32v(1), _i32v(0))
                return run, c0, c1, c2

            z = jnp.zeros((_L,), jnp.int32)
            _, c0, c1, c2 = lax.fori_loop(
                0, Nn, col, (jnp.zeros((_L,), jnp.float32), z, z, z))
            qs = []
            for p, cnt in zip(_PCTS, (c0, c1, c2)):
                idx = jnp.where(cnt >= _i32v(Nn), _i32v(0), cnt)
                yb = plsc.load_gather(cdft, [idx * _L + lane])
                aa = plsc.load_gather(tile, [rowbase + idx])
                qs.append(idx.astype(jnp.float32) + _f32v(1.0) + (_f32v(p) - yb) / aa)
            return qs

        def group(g, acc):
            off = base + g * _L * Nn
            pltpu.sync_copy(pt_hbm.at[pl.ds(off, _L * Nn)], tl_t)
            pltpu.sync_copy(pe_hbm.at[pl.ds(off, _L * Nn)], tl_e)
            q_t = scan_tile(tl_t, cf_t)
            q_e = scan_tile(tl_e, cf_e)
            for k in range(len(_PCTS)):
                acc = acc + jnp.abs(q_t[k] - q_e[k])
            return acc

        acc = lax.fori_loop(0, G, group, jnp.zeros((_L,), jnp.float32))
        v0[...] = acc
        pltpu.sync_copy(v0, out_acc.at[pl.ds(wid * _L, _L)])

    return full


def kernel(p_estimate, p_target):
    Bn, Nn = p_estimate.shape
    we = lax.slice(p_estimate, (0, 0), (Bn, _WIN)).reshape(-1)
    wt = lax.slice(p_target, (0, 0), (Bn, _WIN)).reshape(-1)
    accu = _build_fast(Bn)(we, wt).reshape(_NW, 2, _L)
    loss_fast = jnp.sum(accu[:, 0]) / jnp.float32(Bn * len(_PCTS))
    unres = accu[:, 1]

    def _full_path(ops):
        pe, pt = ops
        facc = _build_full(Bn, Nn)(pe.reshape(-1), pt.reshape(-1))
        return jnp.sum(facc) / jnp.float32(Bn * len(_PCTS))

    def _fast_path(ops):
        return loss_fast

    return lax.cond(jnp.sum(unres) > 0.0, _full_path, _fast_path,
                    (p_estimate, p_target))
